# baseline (speedup 1.0000x reference)
import functools

import jax
import jax.numpy as jnp
from jax import lax
from jax.experimental import pallas as pl
from jax.experimental.pallas import tpu as pltpu
from jax.experimental.pallas import tpu_sc as plsc

_NC, _NS = 2, 16
_NW = _NC * _NS


def kernel(x, emb):
    b, s, d = x.shape
    np_ = b // 2
    pe = emb[:s]
    rows_w = s // _NW
    ch_rows = 8
    n_ch = rows_w // ch_rows

    mesh = plsc.VectorSubcoreMesh(core_axis_name="c", subcore_axis_name="s")

    scratch = (
        [pltpu.VMEM((ch_rows, d), jnp.float32) for _ in range(2)]
        + [pltpu.VMEM((2, ch_rows, d), jnp.float32) for _ in range(2 * np_)]
        + [pltpu.VMEM((2, ch_rows, d), jnp.float32) for _ in range(np_)]
        + [pltpu.SemaphoreType.DMA for _ in range(2 + 3 * np_)]
    )

    @functools.partial(
        pl.kernel,
        out_type=jax.ShapeDtypeStruct((b, s, d), jnp.float32),
        mesh=mesh,
        scratch_types=scratch,
    )
    def sc_add(x_hbm, emb_hbm, out_hbm, *bufs):
        ev = bufs[0:2]
        xv = bufs[2:2 + 2 * np_]
        ov = bufs[2 + 2 * np_:2 + 3 * np_]
        esem = bufs[2 + 3 * np_:4 + 3 * np_]
        xsem = bufs[4 + 3 * np_:4 + 5 * np_]
        osem = bufs[4 + 5 * np_:4 + 6 * np_]

        wid = lax.axis_index("s") * _NC + lax.axis_index("c")
        base = wid * rows_w

        def row(c):
            return base + c * ch_rows

        def load(c, cc, p):
            k = cc * np_ + p
            pltpu.async_copy(
                x_hbm.at[pl.ds(2 * p, 2), pl.ds(row(c), ch_rows)], xv[k], xsem[k]
            )

        def load_wait(c, cc, p):
            k = cc * np_ + p
            pltpu.make_async_copy(
                x_hbm.at[pl.ds(2 * p, 2), pl.ds(row(c), ch_rows)], xv[k], xsem[k]
            ).wait()

        pltpu.async_copy(emb_hbm.at[pl.ds(row(0), ch_rows)], ev[0], esem[0])
        pltpu.async_copy(emb_hbm.at[pl.ds(row(1), ch_rows)], ev[1], esem[1])
        for cc in range(2):
            for p in range(np_):
                load(cc, cc, p)

        @pl.loop(0, n_ch, step=2)
        def _chunks(c0):
            for cc in range(2):
                c = c0 + cc
                for p in range(np_):
                    k = cc * np_ + p
                    load_wait(c, cc, p)
                    if p == 0:
                        pltpu.make_async_copy(
                            emb_hbm.at[pl.ds(row(c), ch_rows)], ev[cc], esem[cc]
                        ).wait()
                    @pl.when(c > 0)
                    def _():
                        pltpu.make_async_copy(
                            ov[p],
                            out_hbm.at[pl.ds(2 * p, 2), pl.ds(row(c), ch_rows)],
                            osem[p],
                        ).wait()

                    @plsc.parallel_loop(0, d, step=16)
                    def _vec(o):
                        for p2 in range(2):
                            for r in range(ch_rows):
                                ov[p][p2, r, pl.ds(o, 16)] = (
                                    xv[k][p2, r, pl.ds(o, 16)]
                                    + ev[cc][r, pl.ds(o, 16)]
                                )

                    @pl.when(c + 2 < n_ch)
                    def _():
                        load(c + 2, cc, p)

                    pltpu.async_copy(
                        ov[p],
                        out_hbm.at[pl.ds(2 * p, 2), pl.ds(row(c), ch_rows)],
                        osem[p],
                    )
                @pl.when(c + 2 < n_ch)
                def _():
                    pltpu.async_copy(
                        emb_hbm.at[pl.ds(row(c + 2), ch_rows)], ev[cc], esem[cc]
                    )

        for p in range(np_):
            pltpu.make_async_copy(
                ov[p],
                out_hbm.at[pl.ds(2 * p, 2), pl.ds(row(n_ch - 1), ch_rows)],
                osem[p],
            ).wait()

    return sc_add(x, pe)


# speedup vs baseline: 1.0116x; 1.0116x over previous
import functools

import jax
import jax.numpy as jnp
from jax import lax
from jax.experimental import pallas as pl
from jax.experimental.pallas import tpu as pltpu
from jax.experimental.pallas import tpu_sc as plsc

_NC, _NS = 2, 16
_NW = _NC * _NS


def kernel(x, emb):
    b, s, d = x.shape
    pe = emb[:s]
    rows_w = s // _NW
    ch_rows = 8
    n_ch = rows_w // ch_rows

    mesh = plsc.VectorSubcoreMesh(core_axis_name="c", subcore_axis_name="s")

    scratch = (
        [pltpu.VMEM((ch_rows, d), jnp.float32) for _ in range(2)]
        + [pltpu.VMEM((ch_rows, d), jnp.float32) for _ in range(2 * b)]
        + [pltpu.VMEM((ch_rows, d), jnp.float32) for _ in range(b)]
        + [pltpu.SemaphoreType.DMA for _ in range(2 + 3 * b)]
    )

    @functools.partial(
        pl.kernel,
        out_type=jax.ShapeDtypeStruct((b, s, d), jnp.float32),
        mesh=mesh,
        scratch_types=scratch,
    )
    def sc_add(x_hbm, emb_hbm, out_hbm, *bufs):
        ev = bufs[0:2]
        xv = bufs[2:2 + 2 * b]
        ov = bufs[2 + 2 * b:2 + 3 * b]
        esem = bufs[2 + 3 * b:4 + 3 * b]
        xsem = bufs[4 + 3 * b:4 + 5 * b]
        osem = bufs[4 + 5 * b:4 + 6 * b]

        wid = lax.axis_index("s") * _NC + lax.axis_index("c")
        base = wid * rows_w

        def row(c):
            return base + c * ch_rows

        def load(c, cc, j):
            k = cc * b + j
            pltpu.async_copy(x_hbm.at[j, pl.ds(row(c), ch_rows)], xv[k], xsem[k])

        def load_wait(c, cc, j):
            k = cc * b + j
            pltpu.make_async_copy(
                x_hbm.at[j, pl.ds(row(c), ch_rows)], xv[k], xsem[k]
            ).wait()

        pltpu.async_copy(emb_hbm.at[pl.ds(row(0), ch_rows)], ev[0], esem[0])
        pltpu.async_copy(emb_hbm.at[pl.ds(row(1), ch_rows)], ev[1], esem[1])
        for cc in range(2):
            for j in range(b):
                load(cc, cc, j)

        @pl.loop(0, n_ch, step=2)
        def _chunks(c0):
            for cc in range(2):
                c = c0 + cc
                for j in range(b):
                    k = cc * b + j
                    load_wait(c, cc, j)
                    if j == 0:
                        pltpu.make_async_copy(
                            emb_hbm.at[pl.ds(row(c), ch_rows)], ev[cc], esem[cc]
                        ).wait()
                    @pl.when(c > 0)
                    def _():
                        pltpu.make_async_copy(
                            ov[j], out_hbm.at[j, pl.ds(row(c), ch_rows)], osem[j]
                        ).wait()

                    @plsc.parallel_loop(0, d, step=16)
                    def _vec(o):
                        for r in range(ch_rows):
                            ov[j][r, pl.ds(o, 16)] = (
                                xv[k][r, pl.ds(o, 16)] + ev[cc][r, pl.ds(o, 16)]
                            )

                    @pl.when(c + 2 < n_ch)
                    def _():
                        load(c + 2, cc, j)

                    pltpu.async_copy(
                        ov[j], out_hbm.at[j, pl.ds(row(c), ch_rows)], osem[j]
                    )
                @pl.when(c + 2 < n_ch)
                def _():
                    pltpu.async_copy(
                        emb_hbm.at[pl.ds(row(c + 2), ch_rows)], ev[cc], esem[cc]
                    )

        for j in range(b):
            pltpu.make_async_copy(
                ov[j], out_hbm.at[j, pl.ds(row(n_ch - 1), ch_rows)], osem[j]
            ).wait()

    return sc_add(x, pe)
